# baseline (speedup 1.0000x reference)
import functools
import numpy as np
import jax
import jax.numpy as jnp
from jax import lax
from jax.experimental import pallas as pl
from jax.experimental.pallas import tpu as pltpu
from jax.experimental.pallas import tpu_sc as plsc

SEG = 8
NVAR = 100
H = 16
D = 1024
G = 10
K = 10
DH = D // H
R = H * G
SCALE = float((D / H) ** -0.5)

NW = 32
CB = 8


def _q_kernel(cl_ref, wq_ref, bq_ref, qblk_ref):
    qq = jnp.dot(cl_ref[0], wq_ref[...], preferred_element_type=jnp.float32)
    qq = qq + bq_ref[...]
    q3 = jnp.broadcast_to(qq[None, :, :], (H, G, D))
    h_iota = lax.broadcasted_iota(jnp.int32, (H, G, D), 0)
    d_iota = lax.broadcasted_iota(jnp.int32, (H, G, D), 2)
    qblk_ref[0] = jnp.where(d_iota // DH == h_iota, q3, 0.0).reshape(R, D)


def _tc1_kernel(x_ref, qblk_ref, wk_ref, bk_ref, wv_ref, bv_ref,
                v_ref, attn_ref, idx_ref, gidx_ref):
    a = x_ref[0]
    a = jnp.log(jnp.maximum(a, 0.0) + 1.0)
    kb = jnp.dot(a, wk_ref[...], preferred_element_type=jnp.float32) + bk_ref[...]
    vb = jnp.dot(a, wv_ref[...], preferred_element_type=jnp.float32) + bv_ref[...]
    v_ref[0] = vb
    qblk = qblk_ref[0]
    scores = lax.dot_general(
        qblk, kb, (((1,), (1,)), ((), ())),
        preferred_element_type=jnp.float32) * SCALE

    lane = lax.broadcasted_iota(jnp.int32, (R, NVAR), 1)
    s = scores
    vals, idxs = [], []
    for _ in range(K):
        m = jnp.max(s, axis=1, keepdims=True)
        cand = jnp.where(s == m, lane, jnp.int32(NVAR))
        j = jnp.min(cand, axis=1, keepdims=True)
        vals.append(m)
        idxs.append(j)
        s = jnp.where(lane == j, -jnp.inf, s)
    topv = jnp.concatenate(vals, axis=1)
    topi = jnp.concatenate(idxs, axis=1)

    e = jnp.exp(topv - topv[:, 0:1])
    attn_ref[0] = e / jnp.sum(e, axis=1, keepdims=True)
    idx_ref[0] = topi
    gidx_ref[0] = jnp.concatenate(
        [topi, jnp.zeros((R, 16 - K), jnp.int32)], axis=1)


def _sc_kernel(idxp_hbm, v_hbm, wg_hbm, bg_hbm, xo_hbm,
               vloc, idxb, outb, wg_v, bg_v):
    c = lax.axis_index("c")
    s = lax.axis_index("s")
    wid = s * 2 + c
    pltpu.sync_copy(wg_hbm, wg_v)
    pltpu.sync_copy(bg_hbm, bg_v)
    bs = v_hbm.shape[0]
    bs_per_w = bs // NW

    def bs_body(t, carry):
        bs_i = wid * bs_per_w + t
        pltpu.sync_copy(idxp_hbm.at[bs_i], idxb)

        def make_sub_body(halfoff):
            def sub_body(sub, carry2):
                for row in range(16):
                    hg = sub * 16 + row
                    h = lax.div(hg, G)
                    g = lax.rem(hg, G)
                    col0 = h * DH - halfoff
                    ivec = idxb[hg]
                    wrow = wg_v[g]
                    bgvec = bg_v[g]
                    accs = []
                    for q in range(4):
                        acc = jnp.zeros((16,), jnp.float32)
                        for r in range(K):
                            acc = acc + vloc[ivec[r], pl.ds(col0 + q * 16, 16)] * wrow[r]
                        accs.append(acc + bgvec)
                    mx4 = jnp.maximum(jnp.maximum(accs[0], accs[1]),
                                      jnp.maximum(accs[2], accs[3]))
                    mn4 = jnp.minimum(jnp.minimum(accs[0], accs[1]),
                                      jnp.minimum(accs[2], accs[3]))
                    mx = lax.reduce_max(mx4, (0,))
                    mn = lax.reduce_min(mn4, (0,))
                    denom = jnp.maximum(mx - mn, 1e-6)
                    dl = g * H + h
                    for q in range(4):
                        outb[dl, pl.ds(q * 16, 16)] = jnp.exp((accs[q] - mn) / denom)
                return carry2
            return sub_body

        pltpu.sync_copy(v_hbm.at[bs_i, :, pl.ds(0, D // 2)], vloc)
        lax.fori_loop(0, 5, make_sub_body(0), 0)
        pltpu.sync_copy(v_hbm.at[bs_i, :, pl.ds(D // 2, D // 2)], vloc)
        lax.fori_loop(5, 10, make_sub_body(D // 2), 0)
        pltpu.sync_copy(outb, xo_hbm.at[pl.ds(bs_i * R, R)])
        return carry

    lax.fori_loop(0, bs_per_w, bs_body, 0)


def _out_kernel(xo_ref, wo_ref, bo_ref, out_ref):
    out_ref[...] = jnp.dot(xo_ref[...], wo_ref[...],
                           preferred_element_type=jnp.float32) + bo_ref[...]


@jax.jit
def _run(x, cluster, Wq, bq, Wk, bk, Wv, bv, Wg, bg, Wo, bo):
    bs = x.shape[0]
    nb = bs // SEG

    qblk = pl.pallas_call(
        _q_kernel,
        grid=(SEG,),
        in_specs=[
            pl.BlockSpec((1, G, D), lambda s: (s, 0, 0)),
            pl.BlockSpec((D, D), lambda s: (0, 0)),
            pl.BlockSpec((1, D), lambda s: (0, 0)),
        ],
        out_specs=pl.BlockSpec((1, R, D), lambda s: (s, 0, 0)),
        out_shape=jax.ShapeDtypeStruct((SEG, R, D), jnp.float32),
    )(cluster, Wq, bq.reshape(1, D))

    v, attn_t, idx_t, gidx = pl.pallas_call(
        _tc1_kernel,
        grid=(SEG, nb),
        in_specs=[
            pl.BlockSpec((1, NVAR, D), lambda s, b: (b * SEG + s, 0, 0)),
            pl.BlockSpec((1, R, D), lambda s, b: (s, 0, 0)),
            pl.BlockSpec((D, D), lambda s, b: (0, 0)),
            pl.BlockSpec((1, D), lambda s, b: (0, 0)),
            pl.BlockSpec((D, D), lambda s, b: (0, 0)),
            pl.BlockSpec((1, D), lambda s, b: (0, 0)),
        ],
        out_specs=[
            pl.BlockSpec((1, NVAR, D), lambda s, b: (b * SEG + s, 0, 0)),
            pl.BlockSpec((1, R, K), lambda s, b: (b * SEG + s, 0, 0)),
            pl.BlockSpec((1, R, K), lambda s, b: (b * SEG + s, 0, 0)),
            pl.BlockSpec((1, R, 16), lambda s, b: (b * SEG + s, 0, 0)),
        ],
        out_shape=[
            jax.ShapeDtypeStruct((bs, NVAR, D), jnp.float32),
            jax.ShapeDtypeStruct((bs, R, K), jnp.float32),
            jax.ShapeDtypeStruct((bs, R, K), jnp.int32),
            jax.ShapeDtypeStruct((bs, R, 16), jnp.int32),
        ],
    )(x, qblk, Wk, bk.reshape(1, D), Wv, bv.reshape(1, D))

    wg_pad = jnp.pad(Wg, ((0, 0), (0, 16 - K)))
    bg_pad = jnp.tile(bg.reshape(G, 1), (1, 16))

    sc = pl.kernel(
        _sc_kernel,
        out_type=jax.ShapeDtypeStruct((bs * R, DH), jnp.float32),
        mesh=plsc.VectorSubcoreMesh(core_axis_name="c", subcore_axis_name="s",
                                    num_cores=2, num_subcores=16),
        compiler_params=pltpu.CompilerParams(needs_layout_passes=False),
        scratch_types=[
            pltpu.VMEM((NVAR, D // 2), jnp.float32),
            pltpu.VMEM((R, 16), jnp.int32),
            pltpu.VMEM((R, DH), jnp.float32),
            pltpu.VMEM((G, 16), jnp.float32),
            pltpu.VMEM((G, 16), jnp.float32),
        ],
    )
    xo = sc(gidx, v, wg_pad, bg_pad)

    xo_mat = xo.reshape(bs * G, D)
    rb = 256 if (bs * G) % 256 == 0 else bs * G
    out = pl.pallas_call(
        _out_kernel,
        grid=((bs * G) // rb,),
        in_specs=[
            pl.BlockSpec((rb, D), lambda i: (i, 0)),
            pl.BlockSpec((D, D), lambda i: (0, 0)),
            pl.BlockSpec((1, D), lambda i: (0, 0)),
        ],
        out_specs=pl.BlockSpec((rb, D), lambda i: (i, 0)),
        out_shape=jax.ShapeDtypeStruct((bs * G, D), jnp.float32),
    )(xo_mat, Wo, bo.reshape(1, D))

    return (out.reshape(bs, G, D),
            attn_t.reshape(bs, H, G, K),
            idx_t.reshape(bs, H, G, K))


def kernel(x, cluster, Wq, bq, Wk, bk, Wv, bv, Wg, bg, Wo, bo):
    return _run(x, cluster, Wq, bq, Wk, bk, Wv, bv, Wg, bg, Wo, bo)


# speedup vs baseline: 1.0980x; 1.0980x over previous
import functools
import numpy as np
import jax
import jax.numpy as jnp
from jax import lax
from jax.experimental import pallas as pl
from jax.experimental.pallas import tpu as pltpu
from jax.experimental.pallas import tpu_sc as plsc

SEG = 8
NVAR = 100
H = 16
D = 1024
G = 10
K = 10
DH = D // H
R = H * G
SCALE = float((D / H) ** -0.5)

NW = 32
CB = 8


def _q_kernel(cl_ref, wq_ref, bq_ref, qblk_ref):
    qq = jnp.dot(cl_ref[0], wq_ref[...], preferred_element_type=jnp.float32)
    qq = qq + bq_ref[...]
    q3 = jnp.broadcast_to(qq[None, :, :], (H, G, D))
    h_iota = lax.broadcasted_iota(jnp.int32, (H, G, D), 0)
    d_iota = lax.broadcasted_iota(jnp.int32, (H, G, D), 2)
    qblk_ref[0] = jnp.where(d_iota // DH == h_iota, q3, 0.0).reshape(R, D)


def _tc1_kernel(x_ref, qblk_ref, wk_ref, bk_ref, wv_ref, bv_ref, p_ref,
                v_ref, attn_ref, idx_ref, sidx_ref):
    bs_i = pl.program_id(1) * SEG + pl.program_id(0)
    a = x_ref[0]
    a = jnp.log(jnp.maximum(a, 0.0) + 1.0)
    kb = jnp.dot(a, wk_ref[...], preferred_element_type=jnp.float32) + bk_ref[...]
    vb = jnp.dot(a, wv_ref[...], preferred_element_type=jnp.float32) + bv_ref[...]
    for j in range(8):
        v_ref[pl.ds(j * NVAR, NVAR)] = vb[:, j * 128:(j + 1) * 128]
    qblk = qblk_ref[0]
    scores = lax.dot_general(
        qblk, kb, (((1,), (1,)), ((), ())),
        preferred_element_type=jnp.float32) * SCALE

    lane = lax.broadcasted_iota(jnp.int32, (R, NVAR), 1)
    s = scores
    vals, idxs = [], []
    for _ in range(K):
        m = jnp.max(s, axis=1, keepdims=True)
        cand = jnp.where(s == m, lane, jnp.int32(NVAR))
        j = jnp.min(cand, axis=1, keepdims=True)
        vals.append(m)
        idxs.append(j)
        s = jnp.where(lane == j, -jnp.inf, s)
    topv = jnp.concatenate(vals, axis=1)
    topi = jnp.concatenate(idxs, axis=1)

    e = jnp.exp(topv - topv[:, 0:1])
    attn_ref[0] = e / jnp.sum(e, axis=1, keepdims=True)
    idx_ref[0] = topi
    hrow = lax.broadcasted_iota(jnp.int32, (R, K), 0) // G
    val = (bs_i * 8 + hrow // 2) * NVAR + topi
    sp = jnp.dot(p_ref[...], val.astype(jnp.float32),
                 preferred_element_type=jnp.float32,
                 precision=lax.Precision.HIGHEST)
    sidx_ref[0] = sp.astype(jnp.int32)


def _sc_kernel(sidx_hbm, v_hbm, wg_hbm, bg_hbm, xo_hbm,
               sidx_all, rows0, rows1, outb, wg_bc, bg_bc, sem0, sem1):
    c = lax.axis_index("c")
    s = lax.axis_index("s")
    wid = s * 2 + c
    pltpu.sync_copy(wg_hbm, wg_bc)
    pltpu.sync_copy(bg_hbm, bg_bc)
    bs = xo_hbm.shape[0] // R
    per_w = (bs * R) // NW
    base = wid * per_w
    nch = per_w // CB
    pch = R // CB
    pltpu.sync_copy(sidx_hbm.at[pl.ds(wid * nch, nch)], sidx_all)

    bufs = (rows0, rows1)
    sems = (sem0, sem1)
    pltpu.async_copy(v_hbm.at[sidx_all.at[0]], rows0, sem0)

    def chunk(t, ph):
        tn = jnp.minimum(t + 1, nch - 1)
        pltpu.async_copy(
            v_hbm.at[sidx_all.at[tn]], bufs[1 - ph], sems[1 - ph])
        pltpu.make_async_copy(
            v_hbm.at[sidx_all.at[t]], bufs[ph], sems[ph]).wait()
        rows = bufs[ph]
        for i in range(CB):
            ridx = base + t * CB + i
            dl = lax.rem(ridx, R)
            g = lax.div(dl, H)
            h = lax.rem(dl, H)
            col0 = lax.rem(h, 2) * DH
            wvecs = [wg_bc[g * K + r] for r in range(K)]
            accs = []
            for q in range(4):
                acc = jnp.zeros((16,), jnp.float32)
                for r in range(K):
                    acc = acc + rows[i * K + r,
                                     pl.ds(col0 + q * 16, 16)] * wvecs[r]
                accs.append(acc + bg_bc[g])
            mx4 = jnp.maximum(jnp.maximum(accs[0], accs[1]),
                              jnp.maximum(accs[2], accs[3]))
            mn4 = jnp.minimum(jnp.minimum(accs[0], accs[1]),
                              jnp.minimum(accs[2], accs[3]))
            mx = lax.reduce_max(mx4, (0,))
            mn = lax.reduce_min(mn4, (0,))
            denom = jnp.maximum(mx - mn, 1e-6)
            lr = lax.rem(t, pch) * CB + i
            for q in range(4):
                outb[lr, pl.ds(q * 16, 16)] = jnp.exp((accs[q] - mn) / denom)
        @pl.when(lax.rem(t, pch) == pch - 1)
        def _():
            bs_i = lax.div(base + t * CB, R)
            pltpu.sync_copy(outb, xo_hbm.at[pl.ds(bs_i * R, R)])

    def body(u, carry):
        chunk(2 * u, 0)
        chunk(2 * u + 1, 1)
        return carry

    lax.fori_loop(0, nch // 2, body, 0)
    pltpu.make_async_copy(
        v_hbm.at[sidx_all.at[nch - 1]], rows0, sem0).wait()


def _out_kernel(xo_ref, wo_ref, bo_ref, out_ref):
    out_ref[...] = jnp.dot(xo_ref[...], wo_ref[...],
                           preferred_element_type=jnp.float32) + bo_ref[...]


@jax.jit
def _run(x, cluster, Wq, bq, Wk, bk, Wv, bv, Wg, bg, Wo, bo):
    bs = x.shape[0]
    nb = bs // SEG

    qblk = pl.pallas_call(
        _q_kernel,
        grid=(SEG,),
        in_specs=[
            pl.BlockSpec((1, G, D), lambda s: (s, 0, 0)),
            pl.BlockSpec((D, D), lambda s: (0, 0)),
            pl.BlockSpec((1, D), lambda s: (0, 0)),
        ],
        out_specs=pl.BlockSpec((1, R, D), lambda s: (s, 0, 0)),
        out_shape=jax.ShapeDtypeStruct((SEG, R, D), jnp.float32),
    )(cluster, Wq, bq.reshape(1, D))

    dl = np.arange(R)
    pmat = np.zeros((R, R), np.float32)
    pmat[dl, (dl % H) * G + dl // H] = 1.0
    perm = jnp.asarray(pmat)
    v, attn_t, idx_t, sidx_t = pl.pallas_call(
        _tc1_kernel,
        grid=(SEG, nb),
        in_specs=[
            pl.BlockSpec((1, NVAR, D), lambda s, b: (b * SEG + s, 0, 0)),
            pl.BlockSpec((1, R, D), lambda s, b: (s, 0, 0)),
            pl.BlockSpec((D, D), lambda s, b: (0, 0)),
            pl.BlockSpec((1, D), lambda s, b: (0, 0)),
            pl.BlockSpec((D, D), lambda s, b: (0, 0)),
            pl.BlockSpec((1, D), lambda s, b: (0, 0)),
            pl.BlockSpec((R, R), lambda s, b: (0, 0)),
        ],
        out_specs=[
            pl.BlockSpec((NVAR * 8, 128), lambda s, b: (b * SEG + s, 0)),
            pl.BlockSpec((1, R, K), lambda s, b: (b * SEG + s, 0, 0)),
            pl.BlockSpec((1, R, K), lambda s, b: (b * SEG + s, 0, 0)),
            pl.BlockSpec((1, R, K), lambda s, b: (b * SEG + s, 0, 0)),
        ],
        out_shape=[
            jax.ShapeDtypeStruct((bs * NVAR * 8, 128), jnp.float32),
            jax.ShapeDtypeStruct((bs, R, K), jnp.float32),
            jax.ShapeDtypeStruct((bs, R, K), jnp.int32),
            jax.ShapeDtypeStruct((bs, R, K), jnp.int32),
        ],
    )(x, qblk, Wk, bk.reshape(1, D), Wv, bv.reshape(1, D), perm)

    wg_bc = jnp.broadcast_to(Wg.reshape(G * K, 1), (G * K, 16))
    bg_bc = jnp.broadcast_to(bg.reshape(G, 1), (G, 16))

    sc = pl.kernel(
        _sc_kernel,
        out_type=jax.ShapeDtypeStruct((bs * R, DH), jnp.float32),
        mesh=plsc.VectorSubcoreMesh(core_axis_name="c", subcore_axis_name="s",
                                    num_cores=2, num_subcores=16),
        compiler_params=pltpu.CompilerParams(needs_layout_passes=False),
        scratch_types=[
            pltpu.VMEM(((bs * R // NW) // CB, CB * K), jnp.int32),
            pltpu.VMEM((CB * K, 128), jnp.float32),
            pltpu.VMEM((CB * K, 128), jnp.float32),
            pltpu.VMEM((R, DH), jnp.float32),
            pltpu.VMEM((G * K, 16), jnp.float32),
            pltpu.VMEM((G, 16), jnp.float32),
            pltpu.SemaphoreType.DMA,
            pltpu.SemaphoreType.DMA,
        ],
    )
    xo = sc(sidx_t.reshape(bs * R * K // (CB * K), CB * K),
            v, wg_bc, bg_bc)

    xo_mat = xo.reshape(bs * G, D)
    rb = 256 if (bs * G) % 256 == 0 else bs * G
    out = pl.pallas_call(
        _out_kernel,
        grid=((bs * G) // rb,),
        in_specs=[
            pl.BlockSpec((rb, D), lambda i: (i, 0)),
            pl.BlockSpec((D, D), lambda i: (0, 0)),
            pl.BlockSpec((1, D), lambda i: (0, 0)),
        ],
        out_specs=pl.BlockSpec((rb, D), lambda i: (i, 0)),
        out_shape=jax.ShapeDtypeStruct((bs * G, D), jnp.float32),
    )(xo_mat, Wo, bo.reshape(1, D))

    return (out.reshape(bs, G, D),
            attn_t.reshape(bs, H, G, K),
            idx_t.reshape(bs, H, G, K))


def kernel(x, cluster, Wq, bq, Wk, bk, Wv, bv, Wg, bg, Wo, bo):
    return _run(x, cluster, Wq, bq, Wk, bk, Wv, bv, Wg, bg, Wo, bo)
